# baseline (speedup 1.0000x reference)
import functools

import jax
import jax.numpy as jnp
from jax import lax
from jax.experimental import pallas as pl
from jax.experimental.pallas import tpu as pltpu
from jax.experimental.pallas import tpu_sc as plsc

B, L = 4096, 200
NTOK = B * L
INTD = 21
D = 32
HID = 64
EPS = 1e-6

NC, NS = 2, 16
NW = NC * NS
TOK_PER_W = NTOK // NW
CH = 256
CHB = CH // 128
NCHUNK = TOK_PER_W // CH
_DO_SUM = True


def _proj_body(e1, e2, e3, e4, w1, w2, w3, w4, p1, p2, p3, p4):
    p1[...] = jnp.dot(e1[...], w1[...], preferred_element_type=jnp.float32)
    p2[...] = jnp.dot(e2[...], w2[...], preferred_element_type=jnp.float32)
    p3[...] = jnp.dot(e3[...], w3[...], preferred_element_type=jnp.float32)
    p4[...] = jnp.dot(e4[...], w4[...], preferred_element_type=jnp.float32)


def _project_tables(E1, E2, E3, E4, W_cat):
    w1, w2, w3, w4 = (W_cat[0:21], W_cat[21:42], W_cat[42:63], W_cat[63:84])
    out_shapes = tuple(
        jax.ShapeDtypeStruct((e.shape[0], D), jnp.float32)
        for e in (E1, E2, E3, E4)
    )
    return pl.pallas_call(
        _proj_body,
        out_shape=out_shapes,
    )(E1, E2, E3, E4, w1, w2, w3, w4)


def _sc_body(ix, p1, p2, p3, p4, out,
             idxv, r1, r2, r3, r4, sv, semi, semg0, semg1, semo0, semo1):
    wid = lax.axis_index("s") * NC + lax.axis_index("c")
    tabs = (p1, p2, p3, p4)
    rbufs = (r1, r2, r3, r4)

    def base(c):
        return wid * TOK_PER_W + c * CH

    def fire_idx(c, b):
        for t in range(4):
            pltpu.async_copy(ix.at[t, pl.ds(base(c), CH)], idxv.at[b, t],
                             semi)

    def wait_idx(b):
        for t in range(4):
            pltpu.make_async_copy(ix.at[0, pl.ds(0, CH)], idxv.at[b, t],
                                  semi).wait()

    def fire_gathers(c, b, semg):
        for t in range(4):
            pltpu.async_copy(tabs[t].at[idxv.at[b, t]], rbufs[t].at[b], semg)

    def wait_gathers(b, semg):
        for t in range(4):
            pltpu.make_async_copy(p1.at[pl.ds(0, CH)], rbufs[t].at[b],
                                  semg).wait()

    def fire_out(c, b, semo):
        pltpu.async_copy(sv.at[b], out.at[pl.ds(base(c), CH)], semo)

    def wait_out(c, b, semo):
        pltpu.make_async_copy(sv.at[b], out.at[pl.ds(base(c), CH)],
                              semo).wait()

    UNROLL = 8

    def do_sum(b):
        def sum_body(t, carry):
            for u in range(UNROLL):
                row = t * UNROLL + u
                for k in range(2):
                    sl = pl.ds(k * 16, 16)
                    sv[b, row, sl] = (r1[b, row, sl] + r2[b, row, sl]
                                      + r3[b, row, sl] + r4[b, row, sl])
            return carry
        lax.fori_loop(0, CH // UNROLL, sum_body, None)

    def chunk(c, b, semg_b, semg_o, semo_b):
        @pl.when(c >= 2)
        def _():
            wait_out(c - 2, b, semo_b)

        @pl.when(c + 1 < NCHUNK)
        def _():
            wait_idx(1 - b)
            fire_gathers(c + 1, 1 - b, semg_o)

        wait_gathers(b, semg_b)

        @pl.when(c + 2 < NCHUNK)
        def _():
            fire_idx(c + 2, b)

        if _DO_SUM:
            do_sum(b)
        fire_out(c, b, semo_b)

    fire_idx(0, 0)
    wait_idx(0)
    fire_gathers(0, 0, semg0)
    fire_idx(1, 1)

    def super_body(i, carry):
        chunk(2 * i, 0, semg0, semg1, semo0)
        chunk(2 * i + 1, 1, semg1, semg0, semo1)
        return carry

    lax.fori_loop(0, NCHUNK // 2, super_body, None)
    wait_out(NCHUNK - 2, 0, semo0)
    wait_out(NCHUNK - 1, 1, semo1)


def _sc_gather_sum(ix, P1, P2, P3, P4):
    mesh = plsc.VectorSubcoreMesh(core_axis_name="c", subcore_axis_name="s")
    fn = functools.partial(
        pl.kernel,
        out_type=jax.ShapeDtypeStruct((NTOK, D), jnp.float32),
        mesh=mesh,
        scratch_types=[
            pltpu.VMEM((2, 4, CH), jnp.int32),
            pltpu.VMEM((2, CH, D), jnp.float32),
            pltpu.VMEM((2, CH, D), jnp.float32),
            pltpu.VMEM((2, CH, D), jnp.float32),
            pltpu.VMEM((2, CH, D), jnp.float32),
            pltpu.VMEM((2, CH, D), jnp.float32),
            pltpu.SemaphoreType.DMA,
            pltpu.SemaphoreType.DMA,
            pltpu.SemaphoreType.DMA,
            pltpu.SemaphoreType.DMA,
            pltpu.SemaphoreType.DMA,
        ],
        compiler_params=pltpu.CompilerParams(use_tc_tiling_on_sc=False),
    )(_sc_body)
    return fn(ix, P1, P2, P3, P4)


def _ln_body(s_ref, x_ref, bcat, gcat, btcat, wnum, bnum, gnum, btnum,
             gout, btout, o_ref):
    s = s_ref[...] + bcat[...]
    mu = jnp.mean(s, axis=-1, keepdims=True)
    xc = s - mu
    var = jnp.mean(xc * xc, axis=-1, keepdims=True)
    cat = xc * lax.rsqrt(var + EPS) * gcat[...] + btcat[...]

    x = x_ref[...]
    h = x * wnum[...] + bnum[...]
    mu2 = jnp.mean(h, axis=-1, keepdims=True)
    hc = h - mu2
    var2 = jnp.mean(hc * hc, axis=-1, keepdims=True)
    num = hc * lax.rsqrt(var2 + EPS) * gnum[...] + btnum[...]

    o = jnp.concatenate([cat, num], axis=-1)
    mu3 = jnp.mean(o, axis=-1, keepdims=True)
    oc = o - mu3
    var3 = jnp.mean(oc * oc, axis=-1, keepdims=True)
    o_ref[...] = oc * lax.rsqrt(var3 + EPS) * gout[...] + btout[...]


def _ln_stage(s, x, b_cat, g_cat, bt_cat, W_num, b_num, g_num, bt_num,
              g_out, bt_out):
    TBLK = 2048
    grid = (NTOK // TBLK,)
    row = lambda i: (i, 0)
    const = lambda i: (0, 0)
    vec32 = pl.BlockSpec((1, D), const)
    vec64 = pl.BlockSpec((1, HID), const)
    return pl.pallas_call(
        _ln_body,
        grid=grid,
        in_specs=[
            pl.BlockSpec((TBLK, D), row),
            pl.BlockSpec((TBLK, 1), row),
            vec32, vec32, vec32, vec32, vec32, vec32, vec32,
            vec64, vec64,
        ],
        out_specs=pl.BlockSpec((TBLK, HID), row),
        out_shape=jax.ShapeDtypeStruct((NTOK, HID), jnp.float32),
    )(s, x,
      b_cat.reshape(1, D), g_cat.reshape(1, D), bt_cat.reshape(1, D),
      W_num.reshape(1, D), b_num.reshape(1, D), g_num.reshape(1, D),
      bt_num.reshape(1, D), g_out.reshape(1, HID), bt_out.reshape(1, HID))


def kernel(test, question, tag, testTag, num_feat,
           E_test, E_q, E_tag, E_tt,
           W_cat, b_cat, g_cat, bt_cat,
           W_num, b_num, g_num, bt_num,
           g_out, bt_out):
    P1, P2, P3, P4 = _project_tables(E_test, E_q, E_tag, E_tt, W_cat)
    idx = lambda a: a.reshape(NTOK)
    ix = jnp.stack([idx(test), idx(question), idx(tag), idx(testTag)], axis=0)
    s = _sc_gather_sum(ix, P1, P2, P3, P4)
    out = _ln_stage(s, num_feat.reshape(NTOK, 1),
                    b_cat, g_cat, bt_cat, W_num, b_num, g_num, bt_num,
                    g_out, bt_out)
    return out.reshape(B, L, HID)


# speedup vs baseline: 3.0416x; 3.0416x over previous
import functools

import jax
import jax.numpy as jnp
from jax import lax
from jax.experimental import pallas as pl
from jax.experimental.pallas import tpu as pltpu
from jax.experimental.pallas import tpu_sc as plsc

B, L = 4096, 200
NTOK = B * L
INTD = 21
D = 32
HID = 64
EPS = 1e-6

NC, NS = 2, 16
NW = NC * NS
TOK_PER_W = NTOK // NW
CH = 256
CHB = CH // 128
NCHUNK = TOK_PER_W // CH
_DO_SUM = True


def _proj_body(e1, e2, e3, e4, w1, w2, w3, w4, p1, p2, p3, p4):
    p1[...] = jnp.dot(e1[...], w1[...], preferred_element_type=jnp.float32)
    p2[...] = jnp.dot(e2[...], w2[...], preferred_element_type=jnp.float32)
    p3[...] = jnp.dot(e3[...], w3[...], preferred_element_type=jnp.float32)
    p4[...] = jnp.dot(e4[...], w4[...], preferred_element_type=jnp.float32)


def _project_tables(E1, E2, E3, E4, W_cat):
    w1, w2, w3, w4 = (W_cat[0:21], W_cat[21:42], W_cat[42:63], W_cat[63:84])
    out_shapes = tuple(
        jax.ShapeDtypeStruct((e.shape[0], D), jnp.float32)
        for e in (E1, E2, E3, E4)
    )
    return pl.pallas_call(
        _proj_body,
        out_shape=out_shapes,
    )(E1, E2, E3, E4, w1, w2, w3, w4)


def _sc_body(ix, p1, p2, p3, p4, out,
             idxv, r1, r2, r3, r4, sv, p1s, p2s, p3s, p4s,
             semi, semg0, semg1, semo0, semo1, semt):
    sid = lax.axis_index("s")
    wid = sid * NC + lax.axis_index("c")
    tabs = (p1s, p2s, p3s, p4s)
    rbufs = (r1, r2, r3, r4)

    @pl.when(sid == 0)
    def _():
        for hb, sh in zip((p1, p2, p3, p4), tabs):
            pltpu.async_copy(hb, sh, semt)
        for hb, sh in zip((p1, p2, p3, p4), tabs):
            pltpu.make_async_copy(hb, sh, semt).wait()

    plsc.subcore_barrier()

    def base(c):
        return wid * TOK_PER_W + c * CH

    def fire_idx(c, b):
        for t in range(4):
            pltpu.async_copy(ix.at[t, pl.ds(base(c), CH)], idxv.at[b, t],
                             semi)

    def wait_idx(b):
        for t in range(4):
            pltpu.make_async_copy(ix.at[0, pl.ds(0, CH)], idxv.at[b, t],
                                  semi).wait()

    def fire_gathers(c, b, semg):
        for t in range(4):
            pltpu.async_copy(tabs[t].at[idxv.at[b, t]], rbufs[t].at[b], semg)

    def wait_gathers(b, semg):
        for t in range(4):
            pltpu.make_async_copy(p1.at[pl.ds(0, CH)], rbufs[t].at[b],
                                  semg).wait()

    def fire_out(c, b, semo):
        pltpu.async_copy(sv.at[b], out.at[pl.ds(base(c), CH)], semo)

    def wait_out(c, b, semo):
        pltpu.make_async_copy(sv.at[b], out.at[pl.ds(base(c), CH)],
                              semo).wait()

    UNROLL = 8

    def do_sum(b):
        def sum_body(t, carry):
            for u in range(UNROLL):
                row = t * UNROLL + u
                for k in range(2):
                    sl = pl.ds(k * 16, 16)
                    sv[b, row, sl] = (r1[b, row, sl] + r2[b, row, sl]
                                      + r3[b, row, sl] + r4[b, row, sl])
            return carry
        lax.fori_loop(0, CH // UNROLL, sum_body, None)

    def chunk(c, b, semg_b, semg_o, semo_b):
        @pl.when(c >= 2)
        def _():
            wait_out(c - 2, b, semo_b)

        @pl.when(c + 1 < NCHUNK)
        def _():
            wait_idx(1 - b)
            fire_gathers(c + 1, 1 - b, semg_o)

        wait_gathers(b, semg_b)

        @pl.when(c + 2 < NCHUNK)
        def _():
            fire_idx(c + 2, b)

        if _DO_SUM:
            do_sum(b)
        fire_out(c, b, semo_b)

    fire_idx(0, 0)
    wait_idx(0)
    fire_gathers(0, 0, semg0)
    fire_idx(1, 1)

    def super_body(i, carry):
        chunk(2 * i, 0, semg0, semg1, semo0)
        chunk(2 * i + 1, 1, semg1, semg0, semo1)
        return carry

    lax.fori_loop(0, NCHUNK // 2, super_body, None)
    wait_out(NCHUNK - 2, 0, semo0)
    wait_out(NCHUNK - 1, 1, semo1)


def _sc_gather_sum(ix, P1, P2, P3, P4):
    mesh = plsc.VectorSubcoreMesh(core_axis_name="c", subcore_axis_name="s")
    fn = functools.partial(
        pl.kernel,
        out_type=jax.ShapeDtypeStruct((NTOK, D), jnp.float32),
        mesh=mesh,
        scratch_types=[
            pltpu.VMEM((2, 4, CH), jnp.int32),
            pltpu.VMEM((2, CH, D), jnp.float32),
            pltpu.VMEM((2, CH, D), jnp.float32),
            pltpu.VMEM((2, CH, D), jnp.float32),
            pltpu.VMEM((2, CH, D), jnp.float32),
            pltpu.VMEM((2, CH, D), jnp.float32),
            pltpu.VMEM_SHARED((1539, D), jnp.float32),
            pltpu.VMEM_SHARED((9456, D), jnp.float32),
            pltpu.VMEM_SHARED((914, D), jnp.float32),
            pltpu.VMEM_SHARED((10, D), jnp.float32),
            pltpu.SemaphoreType.DMA,
            pltpu.SemaphoreType.DMA,
            pltpu.SemaphoreType.DMA,
            pltpu.SemaphoreType.DMA,
            pltpu.SemaphoreType.DMA,
            pltpu.SemaphoreType.DMA,
        ],
        compiler_params=pltpu.CompilerParams(use_tc_tiling_on_sc=False),
    )(_sc_body)
    return fn(ix, P1, P2, P3, P4)


def _ln_body(s_ref, x_ref, bcat, gcat, btcat, wnum, bnum, gnum, btnum,
             gout, btout, o_ref):
    s = s_ref[...] + bcat[...]
    mu = jnp.mean(s, axis=-1, keepdims=True)
    xc = s - mu
    var = jnp.mean(xc * xc, axis=-1, keepdims=True)
    cat = xc * lax.rsqrt(var + EPS) * gcat[...] + btcat[...]

    x = x_ref[...]
    h = x * wnum[...] + bnum[...]
    mu2 = jnp.mean(h, axis=-1, keepdims=True)
    hc = h - mu2
    var2 = jnp.mean(hc * hc, axis=-1, keepdims=True)
    num = hc * lax.rsqrt(var2 + EPS) * gnum[...] + btnum[...]

    o = jnp.concatenate([cat, num], axis=-1)
    mu3 = jnp.mean(o, axis=-1, keepdims=True)
    oc = o - mu3
    var3 = jnp.mean(oc * oc, axis=-1, keepdims=True)
    o_ref[...] = oc * lax.rsqrt(var3 + EPS) * gout[...] + btout[...]


def _ln_stage(s, x, b_cat, g_cat, bt_cat, W_num, b_num, g_num, bt_num,
              g_out, bt_out):
    TBLK = 2048
    grid = (NTOK // TBLK,)
    row = lambda i: (i, 0)
    const = lambda i: (0, 0)
    vec32 = pl.BlockSpec((1, D), const)
    vec64 = pl.BlockSpec((1, HID), const)
    return pl.pallas_call(
        _ln_body,
        grid=grid,
        in_specs=[
            pl.BlockSpec((TBLK, D), row),
            pl.BlockSpec((TBLK, 1), row),
            vec32, vec32, vec32, vec32, vec32, vec32, vec32,
            vec64, vec64,
        ],
        out_specs=pl.BlockSpec((TBLK, HID), row),
        out_shape=jax.ShapeDtypeStruct((NTOK, HID), jnp.float32),
    )(s, x,
      b_cat.reshape(1, D), g_cat.reshape(1, D), bt_cat.reshape(1, D),
      W_num.reshape(1, D), b_num.reshape(1, D), g_num.reshape(1, D),
      bt_num.reshape(1, D), g_out.reshape(1, HID), bt_out.reshape(1, HID))


def kernel(test, question, tag, testTag, num_feat,
           E_test, E_q, E_tag, E_tt,
           W_cat, b_cat, g_cat, bt_cat,
           W_num, b_num, g_num, bt_num,
           g_out, bt_out):
    P1, P2, P3, P4 = _project_tables(E_test, E_q, E_tag, E_tt, W_cat)
    idx = lambda a: a.reshape(NTOK)
    ix = jnp.stack([idx(test), idx(question), idx(tag), idx(testTag)], axis=0)
    s = _sc_gather_sum(ix, P1, P2, P3, P4)
    out = _ln_stage(s, num_feat.reshape(NTOK, 1),
                    b_cat, g_cat, bt_cat, W_num, b_num, g_num, bt_num,
                    g_out, bt_out)
    return out.reshape(B, L, HID)


# speedup vs baseline: 3.4780x; 1.1435x over previous
import functools

import jax
import jax.numpy as jnp
from jax import lax
from jax.experimental import pallas as pl
from jax.experimental.pallas import tpu as pltpu
from jax.experimental.pallas import tpu_sc as plsc

B, L = 4096, 200
NTOK = B * L
INTD = 21
D = 32
HID = 64
EPS = 1e-6

NC, NS = 2, 16
NW = NC * NS
TOK_PER_W = NTOK // NW
CH = 256
CHB = CH // 128
NCHUNK = TOK_PER_W // CH
_DO_SUM = True


def _proj_body(e1, e2, e3, e4, w1, w2, w3, w4, p1, p2, p3, p4):
    p1[...] = jnp.dot(e1[...], w1[...], preferred_element_type=jnp.float32)
    p2[...] = jnp.dot(e2[...], w2[...], preferred_element_type=jnp.float32)
    p3[...] = jnp.dot(e3[...], w3[...], preferred_element_type=jnp.float32)
    p4[...] = jnp.dot(e4[...], w4[...], preferred_element_type=jnp.float32)


def _project_tables(E1, E2, E3, E4, W_cat):
    w1, w2, w3, w4 = (W_cat[0:21], W_cat[21:42], W_cat[42:63], W_cat[63:84])
    out_shapes = tuple(
        jax.ShapeDtypeStruct((e.shape[0], D), jnp.float32)
        for e in (E1, E2, E3, E4)
    )
    return pl.pallas_call(
        _proj_body,
        out_shape=out_shapes,
    )(E1, E2, E3, E4, w1, w2, w3, w4)


def _sc_body(ix, p1, p2, p3, p4, out,
             idxv, r1, r2, r3, r4, sv, p1s, p2s, p3s, p4s,
             semi, semg0, semg1, semo0, semo1, semt):
    sid = lax.axis_index("s")
    wid = sid * NC + lax.axis_index("c")
    tabs = (p1s, p2s, p3s, p4s)
    rbufs = (r1, r2, r3, r4)

    @pl.when(sid == 0)
    def _():
        for hb, sh in zip((p1, p2, p3, p4), tabs):
            pltpu.async_copy(hb, sh, semt)
        for hb, sh in zip((p1, p2, p3, p4), tabs):
            pltpu.make_async_copy(hb, sh, semt).wait()

    plsc.subcore_barrier()

    def base(c):
        return wid * TOK_PER_W + c * CH

    def fire_idx(c, b):
        for t in range(4):
            pltpu.async_copy(ix.at[t, pl.ds(base(c), CH)], idxv.at[b, t],
                             semi)

    def wait_idx(b):
        for t in range(4):
            pltpu.make_async_copy(ix.at[0, pl.ds(0, CH)], idxv.at[b, t],
                                  semi).wait()

    def fire_gathers(c, b, semg):
        for t in range(4):
            pltpu.async_copy(tabs[t].at[idxv.at[b, t]], rbufs[t].at[b], semg)

    def wait_gathers(b, semg):
        for t in range(4):
            pltpu.make_async_copy(p1.at[pl.ds(0, CH)], rbufs[t].at[b],
                                  semg).wait()

    def fire_out(c, b, semo):
        pltpu.async_copy(sv.at[b], out.at[pl.ds(base(c), CH)], semo)

    def wait_out(c, b, semo):
        pltpu.make_async_copy(sv.at[b], out.at[pl.ds(base(c), CH)],
                              semo).wait()

    UNROLL = 8

    def do_sum(b):
        def sum_body(t, carry):
            for u in range(UNROLL):
                row = t * UNROLL + u
                for k in range(2):
                    sl = pl.ds(k * 16, 16)
                    sv[b, row, sl] = (r1[b, row, sl] + r2[b, row, sl]
                                      + r3[b, row, sl] + r4[b, row, sl])
            return carry
        lax.fori_loop(0, CH // UNROLL, sum_body, None)

    def chunk(c, b, semg_b, semg_o, semo_b):
        @pl.when(c >= 2)
        def _():
            wait_out(c - 2, b, semo_b)

        @pl.when(c + 1 < NCHUNK)
        def _():
            wait_idx(1 - b)
            fire_gathers(c + 1, 1 - b, semg_o)

        wait_gathers(b, semg_b)

        @pl.when(c + 2 < NCHUNK)
        def _():
            fire_idx(c + 2, b)

        if _DO_SUM:
            do_sum(b)
        fire_out(c, b, semo_b)

    fire_idx(0, 0)
    wait_idx(0)
    fire_gathers(0, 0, semg0)
    fire_idx(1, 1)

    def super_body(i, carry):
        chunk(2 * i, 0, semg0, semg1, semo0)
        chunk(2 * i + 1, 1, semg1, semg0, semo1)
        return carry

    lax.fori_loop(0, NCHUNK // 2, super_body, None)
    wait_out(NCHUNK - 2, 0, semo0)
    wait_out(NCHUNK - 1, 1, semo1)


def _sc_gather_sum(ix, P1, P2, P3, P4):
    mesh = plsc.VectorSubcoreMesh(core_axis_name="c", subcore_axis_name="s")
    fn = functools.partial(
        pl.kernel,
        out_type=jax.ShapeDtypeStruct((NTOK, D), jnp.float32),
        mesh=mesh,
        scratch_types=[
            pltpu.VMEM((2, 4, CH), jnp.int32),
            pltpu.VMEM((2, CH, D), jnp.float32),
            pltpu.VMEM((2, CH, D), jnp.float32),
            pltpu.VMEM((2, CH, D), jnp.float32),
            pltpu.VMEM((2, CH, D), jnp.float32),
            pltpu.VMEM((2, CH, D), jnp.float32),
            pltpu.VMEM_SHARED((1539, D), jnp.float32),
            pltpu.VMEM_SHARED((9456, D), jnp.float32),
            pltpu.VMEM_SHARED((914, D), jnp.float32),
            pltpu.VMEM_SHARED((10, D), jnp.float32),
            pltpu.SemaphoreType.DMA,
            pltpu.SemaphoreType.DMA,
            pltpu.SemaphoreType.DMA,
            pltpu.SemaphoreType.DMA,
            pltpu.SemaphoreType.DMA,
            pltpu.SemaphoreType.DMA,
        ],
        compiler_params=pltpu.CompilerParams(use_tc_tiling_on_sc=False),
    )(_sc_body)
    return fn(ix, P1, P2, P3, P4)


def _ln_body(s_ref, x_ref, bcat, gcat, btcat, wnum, bnum, gnum, btnum,
             gout, btout, o_ref):
    M32 = jnp.full((D, D), 1.0 / D, dtype=jnp.float32)
    M64 = jnp.full((HID, HID), 1.0 / HID, dtype=jnp.float32)

    s = s_ref[...] + bcat[...]
    mu = jnp.dot(s, M32, preferred_element_type=jnp.float32)
    xc = s - mu
    var = jnp.dot(xc * xc, M32, preferred_element_type=jnp.float32)
    cat = xc * lax.rsqrt(var + EPS) * gcat[...] + btcat[...]

    w = wnum[...]
    bn = bnum[...]
    mw = jnp.mean(w, axis=-1, keepdims=True)
    mb = jnp.mean(bn, axis=-1, keepdims=True)
    wc = w - mw
    bc = bn - mb
    A = jnp.mean(wc * wc, axis=-1, keepdims=True)
    C = jnp.mean(wc * bc, axis=-1, keepdims=True)
    Dq = jnp.mean(bc * bc, axis=-1, keepdims=True)
    x = x_ref[...]
    inv2 = lax.rsqrt(x * x * A + 2.0 * x * C + Dq + EPS)
    num = (x * wc + bc) * inv2 * gnum[...] + btnum[...]

    o = jnp.concatenate([cat, num], axis=-1)
    mu3 = jnp.dot(o, M64, preferred_element_type=jnp.float32)
    oc = o - mu3
    var3 = jnp.dot(oc * oc, M64, preferred_element_type=jnp.float32)
    o_ref[...] = oc * lax.rsqrt(var3 + EPS) * gout[...] + btout[...]


def _ln_stage(s, x, b_cat, g_cat, bt_cat, W_num, b_num, g_num, bt_num,
              g_out, bt_out):
    TBLK = 2048
    grid = (NTOK // TBLK,)
    row = lambda i: (i, 0)
    const = lambda i: (0, 0)
    vec32 = pl.BlockSpec((1, D), const)
    vec64 = pl.BlockSpec((1, HID), const)
    return pl.pallas_call(
        _ln_body,
        grid=grid,
        in_specs=[
            pl.BlockSpec((TBLK, D), row),
            pl.BlockSpec((TBLK, 1), row),
            vec32, vec32, vec32, vec32, vec32, vec32, vec32,
            vec64, vec64,
        ],
        out_specs=pl.BlockSpec((TBLK, HID), row),
        out_shape=jax.ShapeDtypeStruct((NTOK, HID), jnp.float32),
    )(s, x,
      b_cat.reshape(1, D), g_cat.reshape(1, D), bt_cat.reshape(1, D),
      W_num.reshape(1, D), b_num.reshape(1, D), g_num.reshape(1, D),
      bt_num.reshape(1, D), g_out.reshape(1, HID), bt_out.reshape(1, HID))


def kernel(test, question, tag, testTag, num_feat,
           E_test, E_q, E_tag, E_tt,
           W_cat, b_cat, g_cat, bt_cat,
           W_num, b_num, g_num, bt_num,
           g_out, bt_out):
    P1, P2, P3, P4 = _project_tables(E_test, E_q, E_tag, E_tt, W_cat)
    idx = lambda a: a.reshape(NTOK)
    ix = jnp.stack([idx(test), idx(question), idx(tag), idx(testTag)], axis=0)
    s = _sc_gather_sum(ix, P1, P2, P3, P4)
    out = _ln_stage(s, num_feat.reshape(NTOK, 1),
                    b_cat, g_cat, bt_cat, W_num, b_num, g_num, bt_num,
                    g_out, bt_out)
    return out.reshape(B, L, HID)


# speedup vs baseline: 3.8876x; 1.1177x over previous
import functools

import jax
import jax.numpy as jnp
from jax import lax
from jax.experimental import pallas as pl
from jax.experimental.pallas import tpu as pltpu
from jax.experimental.pallas import tpu_sc as plsc

B, L = 4096, 200
NTOK = B * L
INTD = 21
D = 32
HID = 64
EPS = 1e-6

NC, NS = 2, 16
NW = NC * NS
TOK_PER_W = NTOK // NW
CH = 256
CHB = CH // 128
NCHUNK = TOK_PER_W // CH
_DO_SUM = True


def _proj_body(e1, e2, e3, e4, w1, w2, w3, w4, p1, p2, p3, p4):
    p1[...] = jnp.dot(e1[...], w1[...], preferred_element_type=jnp.float32)
    p2[...] = jnp.dot(e2[...], w2[...], preferred_element_type=jnp.float32)
    p3[...] = jnp.dot(e3[...], w3[...], preferred_element_type=jnp.float32)
    p4[...] = jnp.dot(e4[...], w4[...], preferred_element_type=jnp.float32)


def _project_tables(E1, E2, E3, E4, W_cat):
    w1, w2, w3, w4 = (W_cat[0:21], W_cat[21:42], W_cat[42:63], W_cat[63:84])
    out_shapes = tuple(
        jax.ShapeDtypeStruct((e.shape[0], D), jnp.float32)
        for e in (E1, E2, E3, E4)
    )
    return pl.pallas_call(
        _proj_body,
        out_shape=out_shapes,
    )(E1, E2, E3, E4, w1, w2, w3, w4)


def _sc_body(ix, p1, p2, p3, p4, out,
             idxv, r1, r2, r3, r4, sv, p1s, p2s, p3s, p4s,
             semi, semg0, semg1, semo0, semo1, semt):
    sid = lax.axis_index("s")
    wid = sid * NC + lax.axis_index("c")
    tabs = (p1s, p2s, p3s, p4s)
    rbufs = (r1, r2, r3, r4)

    @pl.when(sid == 0)
    def _():
        for hb, sh in zip((p1, p2, p3, p4), tabs):
            pltpu.async_copy(hb, sh, semt)
        for hb, sh in zip((p1, p2, p3, p4), tabs):
            pltpu.make_async_copy(hb, sh, semt).wait()

    plsc.subcore_barrier()

    def base(c):
        return wid * TOK_PER_W + c * CH

    def fire_idx(c, b):
        for t in range(4):
            pltpu.async_copy(ix.at[t, pl.ds(base(c), CH)], idxv.at[b, t],
                             semi)

    def wait_idx(b):
        for t in range(4):
            pltpu.make_async_copy(ix.at[0, pl.ds(0, CH)], idxv.at[b, t],
                                  semi).wait()

    def fire_gathers(c, b, semg):
        for t in range(4):
            pltpu.async_copy(tabs[t].at[idxv.at[b, t]], rbufs[t].at[b], semg)

    def wait_gathers(b, semg):
        for t in range(4):
            pltpu.make_async_copy(p1.at[pl.ds(0, CH)], rbufs[t].at[b],
                                  semg).wait()

    def fire_out(c, b, semo):
        pltpu.async_copy(sv.at[b], out.at[pl.ds(base(c) // 4, CH // 4)],
                         semo)

    def wait_out(c, b, semo):
        pltpu.make_async_copy(sv.at[b], out.at[pl.ds(base(c) // 4, CH // 4)],
                              semo).wait()

    UNROLL = 8

    def do_sum(b):
        def sum_body(t, carry):
            for u in range(UNROLL):
                row = t * UNROLL + u
                prow = 2 * t + u // 4
                for k in range(2):
                    sl = pl.ds(k * 16, 16)
                    psl = pl.ds((u % 4) * 32 + k * 16, 16)
                    sv[b, prow, psl] = (r1[b, row, sl] + r2[b, row, sl]
                                        + r3[b, row, sl] + r4[b, row, sl])
            return carry
        lax.fori_loop(0, CH // UNROLL, sum_body, None)

    def chunk(c, b, semg_b, semg_o, semo_b):
        @pl.when(c >= 2)
        def _():
            wait_out(c - 2, b, semo_b)

        @pl.when(c + 1 < NCHUNK)
        def _():
            wait_idx(1 - b)
            fire_gathers(c + 1, 1 - b, semg_o)

        wait_gathers(b, semg_b)

        @pl.when(c + 2 < NCHUNK)
        def _():
            fire_idx(c + 2, b)

        if _DO_SUM:
            do_sum(b)
        fire_out(c, b, semo_b)

    fire_idx(0, 0)
    wait_idx(0)
    fire_gathers(0, 0, semg0)
    fire_idx(1, 1)

    def super_body(i, carry):
        chunk(2 * i, 0, semg0, semg1, semo0)
        chunk(2 * i + 1, 1, semg1, semg0, semo1)
        return carry

    lax.fori_loop(0, NCHUNK // 2, super_body, None)
    wait_out(NCHUNK - 2, 0, semo0)
    wait_out(NCHUNK - 1, 1, semo1)


def _sc_gather_sum(ix, P1, P2, P3, P4):
    mesh = plsc.VectorSubcoreMesh(core_axis_name="c", subcore_axis_name="s")
    fn = functools.partial(
        pl.kernel,
        out_type=jax.ShapeDtypeStruct((NTOK // 4, 128), jnp.float32),
        mesh=mesh,
        scratch_types=[
            pltpu.VMEM((2, 4, CH), jnp.int32),
            pltpu.VMEM((2, CH, D), jnp.float32),
            pltpu.VMEM((2, CH, D), jnp.float32),
            pltpu.VMEM((2, CH, D), jnp.float32),
            pltpu.VMEM((2, CH, D), jnp.float32),
            pltpu.VMEM((2, CH // 4, 128), jnp.float32),
            pltpu.VMEM_SHARED((1539, D), jnp.float32),
            pltpu.VMEM_SHARED((9456, D), jnp.float32),
            pltpu.VMEM_SHARED((914, D), jnp.float32),
            pltpu.VMEM_SHARED((10, D), jnp.float32),
            pltpu.SemaphoreType.DMA,
            pltpu.SemaphoreType.DMA,
            pltpu.SemaphoreType.DMA,
            pltpu.SemaphoreType.DMA,
            pltpu.SemaphoreType.DMA,
            pltpu.SemaphoreType.DMA,
        ],
        compiler_params=pltpu.CompilerParams(use_tc_tiling_on_sc=False),
    )(_sc_body)
    return fn(ix, P1, P2, P3, P4)


TBLK4 = 512


def _ln_body(s_ref, x_ref, R32, R64, Ex, SCe, SNe, SCo, SNo,
             bcat, gcat, btcat, wc4, bc4, acd, gnum, btnum,
             goc, gon, btoc, bton, o_ref):
    f32 = jnp.float32
    s = s_ref[...] + bcat[...]
    mu = jnp.dot(s, R32[...], preferred_element_type=f32)
    xc = s - mu
    var = jnp.dot(xc * xc, R32[...], preferred_element_type=f32)
    cat = xc * lax.rsqrt(var + EPS) * gcat[...] + btcat[...]

    xe = jnp.dot(Ex[...], x_ref[...], preferred_element_type=f32)
    ii = lax.broadcasted_iota(jnp.int32, (TBLK4, 128), 0)
    jj = lax.broadcasted_iota(jnp.int32, (TBLK4, 128), 1)
    x = jnp.take_along_axis(xe, 4 * (ii % 32) + jj // 32, axis=1)
    A = acd[0, 0]
    C = acd[0, 1]
    Dq = acd[0, 2]
    inv2 = lax.rsqrt(x * x * A + 2.0 * C * x + Dq + EPS)
    num = (x * wc4[...] + bc4[...]) * inv2 * gnum[...] + btnum[...]

    mu3 = jnp.dot(cat + num, R64[...], preferred_element_type=f32)
    cc = cat - mu3
    nc = num - mu3
    var3 = jnp.dot(cc * cc + nc * nc, R64[...], preferred_element_type=f32)
    r3 = lax.rsqrt(var3 + EPS)
    oc = cc * r3 * goc[...] + btoc[...]
    on = nc * r3 * gon[...] + bton[...]
    o_ref[:, 0, :] = (jnp.dot(oc, SCe[...], preferred_element_type=f32)
                      + jnp.dot(on, SNe[...], preferred_element_type=f32))
    o_ref[:, 1, :] = (jnp.dot(oc, SCo[...], preferred_element_type=f32)
                      + jnp.dot(on, SNo[...], preferred_element_type=f32))


def _sel(pairs):
    m = jnp.zeros((128, 128), dtype=jnp.float32)
    i = lax.broadcasted_iota(jnp.int32, (128, 128), 0)
    j = lax.broadcasted_iota(jnp.int32, (128, 128), 1)
    for src, dst in pairs:
        m = m + jnp.where((j >= dst) & (j < dst + 32) & (i == j - dst + src),
                          1.0, 0.0)
    return m


def _ln_stage(s, x, b_cat, g_cat, bt_cat, W_num, b_num, g_num, bt_num,
              g_out, bt_out):
    i = lax.broadcasted_iota(jnp.int32, (128, 128), 0)
    j = lax.broadcasted_iota(jnp.int32, (128, 128), 1)
    seg = jnp.where(i // 32 == j // 32, 1.0, 0.0)
    R32 = seg / 32.0
    R64 = seg / 64.0
    Ex = jnp.where(lax.broadcasted_iota(jnp.int32, (TBLK4, 16), 0) // 32
                   == lax.broadcasted_iota(jnp.int32, (TBLK4, 16), 1),
                   1.0, 0.0).astype(jnp.float32)
    SCe = _sel([(0, 0), (32, 64)])
    SNe = _sel([(0, 32), (32, 96)])
    SCo = _sel([(64, 0), (96, 64)])
    SNo = _sel([(64, 32), (96, 96)])

    w = W_num.reshape(D)
    bn = b_num
    wc = w - jnp.mean(w)
    bc = bn - jnp.mean(bn)
    acd = jnp.stack([jnp.mean(wc * wc), jnp.mean(wc * bc),
                     jnp.mean(bc * bc)]).reshape(1, 3)
    t4 = lambda v: jnp.tile(v, 4).reshape(1, 128)

    grid = (NTOK // 4 // TBLK4,)
    row = lambda g: (g, 0)
    const2 = lambda g: (0, 0)
    mat = pl.BlockSpec((128, 128), const2)
    vec = pl.BlockSpec((1, 128), const2)
    return pl.pallas_call(
        _ln_body,
        grid=grid,
        in_specs=[
            pl.BlockSpec((TBLK4, 128), row),
            pl.BlockSpec((TBLK4 // 32, 128), row),
            mat, mat, pl.BlockSpec((TBLK4, 16), const2),
            mat, mat, mat, mat,
            vec, vec, vec, vec, vec,
            pl.BlockSpec((1, 3), const2),
            vec, vec, vec, vec, vec, vec,
        ],
        out_specs=pl.BlockSpec((TBLK4, 2, 128), lambda g: (g, 0, 0)),
        out_shape=jax.ShapeDtypeStruct((NTOK // 4, 2, 128), jnp.float32),
    )(s, x, R32, R64, Ex, SCe, SNe, SCo, SNo,
      t4(b_cat), t4(g_cat), t4(bt_cat), t4(wc), t4(bc), acd,
      t4(g_num), t4(bt_num),
      t4(g_out[0:D]), t4(g_out[D:]), t4(bt_out[0:D]), t4(bt_out[D:]))


def kernel(test, question, tag, testTag, num_feat,
           E_test, E_q, E_tag, E_tt,
           W_cat, b_cat, g_cat, bt_cat,
           W_num, b_num, g_num, bt_num,
           g_out, bt_out):
    P1, P2, P3, P4 = _project_tables(E_test, E_q, E_tag, E_tt, W_cat)
    idx = lambda a: a.reshape(NTOK)
    ix = jnp.stack([idx(test), idx(question), idx(tag), idx(testTag)], axis=0)
    s = _sc_gather_sum(ix, P1, P2, P3, P4)
    out = _ln_stage(s, num_feat.reshape(NTOK // 128, 128),
                    b_cat, g_cat, bt_cat, W_num, b_num, g_num, bt_num,
                    g_out, bt_out)
    return out.reshape(B, L, HID)
